# baseline (speedup 1.0000x reference)
import functools

import jax
import jax.numpy as jnp
from jax import lax
from jax.experimental import pallas as pl
from jax.experimental.pallas import tpu as pltpu
from jax.experimental.pallas import tpu_sc as plsc

NUM_TOKENS = 16384
D_HIDDEN = 4096
NUM_EXPERTS = 64
TOP_K = 8
BLK = 512

_NC = 2
_NS = 16
_NW = _NC * _NS
_TPW = NUM_TOKENS // _NW


def _dense_body(x_ref, w_ref, scores_ref):
    x = x_ref[...]
    w = w_ref[...]
    logits = lax.dot_general(
        x, w, (((1,), (1,)), ((), ())), preferred_element_type=jnp.float32
    )
    m = jnp.max(logits, axis=1, keepdims=True)
    e = jnp.exp(logits - m)
    s = jnp.sum(e, axis=1, keepdims=True)
    scores_ref[...] = e / s


def _dense(x, W_g):
    return pl.pallas_call(
        _dense_body,
        grid=(NUM_TOKENS // BLK,),
        in_specs=[
            pl.BlockSpec((BLK, D_HIDDEN), lambda i: (i, 0)),
            pl.BlockSpec((NUM_EXPERTS, D_HIDDEN), lambda i: (0, 0)),
        ],
        out_specs=pl.BlockSpec((BLK, NUM_EXPERTS), lambda i: (i, 0)),
        out_shape=jax.ShapeDtypeStruct((NUM_TOKENS, NUM_EXPERTS), jnp.float32),
    )(x, W_g)


def _sc_topk_body(scores_hbm, idx_hbm, tks_hbm, scores_v, idx_v, tks_v):
    wid = lax.axis_index("s") * _NC + lax.axis_index("c")
    pltpu.sync_copy(
        scores_hbm.at[pl.ds(wid * (_TPW * NUM_EXPERTS), _TPW * NUM_EXPERTS)],
        scores_v,
    )

    iota = lax.iota(jnp.int32, 16)
    lm = iota < 8

    def body(t, carry):
        base = t * NUM_EXPERTS
        s0 = scores_v[pl.ds(base, 16)]
        s1 = scores_v[pl.ds(base + 16, 16)]
        s2 = scores_v[pl.ds(base + 32, 16)]
        s3 = scores_v[pl.ds(base + 48, 16)]
        k0, v0 = plsc.sort_key_val(s0, iota, descending=True)
        k1, v1 = plsc.sort_key_val(s1, iota + 16, descending=False)
        k2, v2 = plsc.sort_key_val(s2, iota + 32, descending=True)
        k3, v3 = plsc.sort_key_val(s3, iota + 48, descending=False)
        c1k = jnp.where(lm, k0, k1)
        c1v = jnp.where(lm, v0, v1)
        c2k = jnp.where(lm, k2, k3)
        c2v = jnp.where(lm, v2, v3)
        d1k, d1v = plsc.sort_key_val(c1k, c1v, descending=True)
        d2k, d2v = plsc.sort_key_val(c2k, c2v, descending=False)
        fk0 = jnp.where(lm, d1k, d2k)
        fv0 = jnp.where(lm, d1v, d2v)
        fk, fv = plsc.sort_key_val(fk0, fv0, descending=True)
        ssum = jnp.sum(jnp.where(lm, fk, 0.0), axis=0)
        tks = fk / ssum
        plsc.store_scatter(idx_v, [t * TOP_K + iota], fv, mask=lm)
        plsc.store_scatter(tks_v, [t * TOP_K + iota], tks, mask=lm)
        return carry

    lax.fori_loop(0, _TPW, body, 0, unroll=4)

    pltpu.sync_copy(idx_v, idx_hbm.at[pl.ds(wid * (_TPW * TOP_K), _TPW * TOP_K)])
    pltpu.sync_copy(tks_v, tks_hbm.at[pl.ds(wid * (_TPW * TOP_K), _TPW * TOP_K)])


@functools.partial(
    pl.kernel,
    mesh=plsc.VectorSubcoreMesh(core_axis_name="c", subcore_axis_name="s"),
    out_type=(
        jax.ShapeDtypeStruct((NUM_TOKENS * TOP_K,), jnp.int32),
        jax.ShapeDtypeStruct((NUM_TOKENS * TOP_K,), jnp.float32),
    ),
    scratch_types=[
        pltpu.VMEM((_TPW * NUM_EXPERTS,), jnp.float32),
        pltpu.VMEM((_TPW * TOP_K,), jnp.int32),
        pltpu.VMEM((_TPW * TOP_K,), jnp.float32),
    ],
    compiler_params=pltpu.CompilerParams(needs_layout_passes=False),
)
def _sc_topk(scores_hbm, idx_hbm, tks_hbm, scores_v, idx_v, tks_v):
    _sc_topk_body(scores_hbm, idx_hbm, tks_hbm, scores_v, idx_v, tks_v)


def kernel(x, W_g):
    scores = _dense(x, W_g)
    idx_flat, tks_flat = _sc_topk(scores.reshape(-1))
    return (
        idx_flat.reshape(NUM_TOKENS, TOP_K),
        tks_flat.reshape(NUM_TOKENS, TOP_K),
        scores,
    )


# speedup vs baseline: 1.1402x; 1.1402x over previous
import functools

import jax
import jax.numpy as jnp
from jax import lax
from jax.experimental import pallas as pl
from jax.experimental.pallas import tpu as pltpu
from jax.experimental.pallas import tpu_sc as plsc

NUM_TOKENS = 16384
D_HIDDEN = 4096
NUM_EXPERTS = 64
TOP_K = 8
BLK = 512

_NC = 2
_NS = 16
_NW = _NC * _NS
_TPW = NUM_TOKENS // _NW


def _dense_body(x_ref, w_ref, scores_ref):
    x = x_ref[...]
    w = w_ref[...]
    logits = lax.dot_general(
        x, w, (((1,), (1,)), ((), ())), preferred_element_type=jnp.float32
    )
    m = jnp.max(logits, axis=1, keepdims=True)
    e = jnp.exp(logits - m)
    s = jnp.sum(e, axis=1, keepdims=True)
    scores_ref[...] = e / s


def _dense(x, W_g):
    return pl.pallas_call(
        _dense_body,
        grid=(NUM_TOKENS // BLK,),
        in_specs=[
            pl.BlockSpec((BLK, D_HIDDEN), lambda i: (i, 0)),
            pl.BlockSpec((NUM_EXPERTS, D_HIDDEN), lambda i: (0, 0)),
        ],
        out_specs=pl.BlockSpec((BLK, NUM_EXPERTS), lambda i: (i, 0)),
        out_shape=jax.ShapeDtypeStruct((NUM_TOKENS, NUM_EXPERTS), jnp.float32),
    )(x, W_g)


def _sc_topk_body(scores_hbm, idx_hbm, tks_hbm, scores_v, idx_v, tks_v):
    wid = lax.axis_index("s") * _NC + lax.axis_index("c")
    pltpu.sync_copy(
        scores_hbm.at[pl.ds(wid * (_TPW * NUM_EXPERTS), _TPW * NUM_EXPERTS)],
        scores_v,
    )

    iota = lax.iota(jnp.int32, 16)
    lm = iota < 8

    @plsc.parallel_loop(0, _TPW, unroll=8)
    def body(t):
        base = t * NUM_EXPERTS
        s0 = scores_v[pl.ds(base, 16)]
        s1 = scores_v[pl.ds(base + 16, 16)]
        s2 = scores_v[pl.ds(base + 32, 16)]
        s3 = scores_v[pl.ds(base + 48, 16)]
        k0, v0 = plsc.sort_key_val(s0, iota, descending=True)
        k1, v1 = plsc.sort_key_val(s1, iota + 16, descending=False)
        k2, v2 = plsc.sort_key_val(s2, iota + 32, descending=True)
        k3, v3 = plsc.sort_key_val(s3, iota + 48, descending=False)
        c1k = jnp.where(lm, k0, k1)
        c1v = jnp.where(lm, v0, v1)
        c2k = jnp.where(lm, k2, k3)
        c2v = jnp.where(lm, v2, v3)
        d1k, d1v = plsc.sort_key_val(c1k, c1v, descending=True)
        d2k, d2v = plsc.sort_key_val(c2k, c2v, descending=False)
        fk0 = jnp.where(lm, d1k, d2k)
        fv0 = jnp.where(lm, d1v, d2v)
        fk, fv = plsc.sort_key_val(fk0, fv0, descending=True)
        ssum = jnp.sum(jnp.where(lm, fk, 0.0), axis=0)
        tks = fk / ssum
        plsc.store_scatter(idx_v, [t * TOP_K + iota], fv, mask=lm)
        plsc.store_scatter(tks_v, [t * TOP_K + iota], tks, mask=lm)

    pltpu.sync_copy(idx_v, idx_hbm.at[pl.ds(wid * (_TPW * TOP_K), _TPW * TOP_K)])
    pltpu.sync_copy(tks_v, tks_hbm.at[pl.ds(wid * (_TPW * TOP_K), _TPW * TOP_K)])


@functools.partial(
    pl.kernel,
    mesh=plsc.VectorSubcoreMesh(core_axis_name="c", subcore_axis_name="s"),
    out_type=(
        jax.ShapeDtypeStruct((NUM_TOKENS * TOP_K,), jnp.int32),
        jax.ShapeDtypeStruct((NUM_TOKENS * TOP_K,), jnp.float32),
    ),
    scratch_types=[
        pltpu.VMEM((_TPW * NUM_EXPERTS,), jnp.float32),
        pltpu.VMEM((_TPW * TOP_K,), jnp.int32),
        pltpu.VMEM((_TPW * TOP_K,), jnp.float32),
    ],
    compiler_params=pltpu.CompilerParams(needs_layout_passes=False),
)
def _sc_topk(scores_hbm, idx_hbm, tks_hbm, scores_v, idx_v, tks_v):
    _sc_topk_body(scores_hbm, idx_hbm, tks_hbm, scores_v, idx_v, tks_v)


def kernel(x, W_g):
    scores = _dense(x, W_g)
    idx_flat, tks_flat = _sc_topk(scores.reshape(-1))
    return (
        idx_flat.reshape(NUM_TOKENS, TOP_K),
        tks_flat.reshape(NUM_TOKENS, TOP_K),
        scores,
    )


# speedup vs baseline: 1.1449x; 1.0041x over previous
import functools

import jax
import jax.numpy as jnp
from jax import lax
from jax.experimental import pallas as pl
from jax.experimental.pallas import tpu as pltpu
from jax.experimental.pallas import tpu_sc as plsc

NUM_TOKENS = 16384
D_HIDDEN = 4096
NUM_EXPERTS = 64
TOP_K = 8
BLK = 1024

_NC = 2
_NS = 16
_NW = _NC * _NS
_TPW = NUM_TOKENS // _NW


def _dense_body(x_ref, w_ref, scores_ref):
    x = x_ref[...]
    w = w_ref[...]
    logits = lax.dot_general(
        x, w, (((1,), (1,)), ((), ())), preferred_element_type=jnp.float32
    )
    m = jnp.max(logits, axis=1, keepdims=True)
    e = jnp.exp(logits - m)
    s = jnp.sum(e, axis=1, keepdims=True)
    scores_ref[...] = e / s


def _dense(x, W_g):
    return pl.pallas_call(
        _dense_body,
        grid=(NUM_TOKENS // BLK,),
        in_specs=[
            pl.BlockSpec((BLK, D_HIDDEN), lambda i: (i, 0)),
            pl.BlockSpec((NUM_EXPERTS, D_HIDDEN), lambda i: (0, 0)),
        ],
        out_specs=pl.BlockSpec((BLK, NUM_EXPERTS), lambda i: (i, 0)),
        out_shape=jax.ShapeDtypeStruct((NUM_TOKENS, NUM_EXPERTS), jnp.float32),
    )(x, W_g)


def _sc_topk_body(scores_hbm, idx_hbm, tks_hbm, scores_v, idx_v, tks_v):
    wid = lax.axis_index("s") * _NC + lax.axis_index("c")
    pltpu.sync_copy(
        scores_hbm.at[pl.ds(wid * (_TPW * NUM_EXPERTS), _TPW * NUM_EXPERTS)],
        scores_v,
    )

    iota = lax.iota(jnp.int32, 16)
    lm = iota < 8

    @plsc.parallel_loop(0, _TPW, unroll=8)
    def body(t):
        base = t * NUM_EXPERTS
        s0 = scores_v[pl.ds(base, 16)]
        s1 = scores_v[pl.ds(base + 16, 16)]
        s2 = scores_v[pl.ds(base + 32, 16)]
        s3 = scores_v[pl.ds(base + 48, 16)]
        k0, v0 = plsc.sort_key_val(s0, iota, descending=True)
        k1, v1 = plsc.sort_key_val(s1, iota + 16, descending=False)
        k2, v2 = plsc.sort_key_val(s2, iota + 32, descending=True)
        k3, v3 = plsc.sort_key_val(s3, iota + 48, descending=False)
        c1k = jnp.where(lm, k0, k1)
        c1v = jnp.where(lm, v0, v1)
        c2k = jnp.where(lm, k2, k3)
        c2v = jnp.where(lm, v2, v3)
        d1k, d1v = plsc.sort_key_val(c1k, c1v, descending=True)
        d2k, d2v = plsc.sort_key_val(c2k, c2v, descending=False)
        fk0 = jnp.where(lm, d1k, d2k)
        fv0 = jnp.where(lm, d1v, d2v)
        fk, fv = plsc.sort_key_val(fk0, fv0, descending=True)
        ssum = jnp.sum(jnp.where(lm, fk, 0.0), axis=0)
        tks = fk / ssum
        plsc.store_scatter(idx_v, [t * TOP_K + iota], fv, mask=lm)
        plsc.store_scatter(tks_v, [t * TOP_K + iota], tks, mask=lm)

    pltpu.sync_copy(idx_v, idx_hbm.at[pl.ds(wid * (_TPW * TOP_K), _TPW * TOP_K)])
    pltpu.sync_copy(tks_v, tks_hbm.at[pl.ds(wid * (_TPW * TOP_K), _TPW * TOP_K)])


@functools.partial(
    pl.kernel,
    mesh=plsc.VectorSubcoreMesh(core_axis_name="c", subcore_axis_name="s"),
    out_type=(
        jax.ShapeDtypeStruct((NUM_TOKENS * TOP_K,), jnp.int32),
        jax.ShapeDtypeStruct((NUM_TOKENS * TOP_K,), jnp.float32),
    ),
    scratch_types=[
        pltpu.VMEM((_TPW * NUM_EXPERTS,), jnp.float32),
        pltpu.VMEM((_TPW * TOP_K,), jnp.int32),
        pltpu.VMEM((_TPW * TOP_K,), jnp.float32),
    ],
    compiler_params=pltpu.CompilerParams(needs_layout_passes=False),
)
def _sc_topk(scores_hbm, idx_hbm, tks_hbm, scores_v, idx_v, tks_v):
    _sc_topk_body(scores_hbm, idx_hbm, tks_hbm, scores_v, idx_v, tks_v)


def kernel(x, W_g):
    scores = _dense(x, W_g)
    idx_flat, tks_flat = _sc_topk(scores.reshape(-1))
    return (
        idx_flat.reshape(NUM_TOKENS, TOP_K),
        tks_flat.reshape(NUM_TOKENS, TOP_K),
        scores,
    )
